# baseline (speedup 1.0000x reference)
import jax
import jax.numpy as jnp
import numpy as np
from jax.experimental import pallas as pl
from jax.experimental.pallas import tpu as pltpu

_G = 4
_TLEN = 16
_NN = 16
_D2 = 16
_HID = 8
_EPS = 1e-5
_DECAY = 0.7
_NEG = 0.01
_R = 256


def _leaky(x):
    return jnp.where(x >= 0, x, _NEG * x)


def _body(xu_ref, pe_ref, m1_ref, b1_ref, m2_ref, b2_ref, l2_ref, l2b_ref,
          g1t_ref, g1b_ref, g2t_ref, g2b_ref,
          tf1_ref, c1_ref, tf2_ref, c2_ref,
          m1s_ref, m1b_ref, m2s_ref, m2b_ref,
          en1_ref, w1m_ref, mx1_ref,
          dn2_ref, w2m_ref, mx2_ref,
          wp1_ref, wp2_ref, fb1_ref,
          f2t_ref, fb2_ref, f3t_ref, fb3_ref, f4_ref, fb4_ref,
          out_ref):
    f32 = jnp.float32
    x = xu_ref[...]
    h = jnp.maximum(jnp.dot(x, m1_ref[...], preferred_element_type=f32)
                    + b1_ref[...], 0.0)
    h = jnp.maximum(jnp.dot(h, m2_ref[...], preferred_element_type=f32)
                    + b2_ref[...], 0.0)
    a4 = (jnp.dot(h, l2_ref[...], preferred_element_type=f32)
          + l2b_ref[...] + pe_ref[...])

    nf1 = jnp.dot(a4, g1t_ref[...], preferred_element_type=f32) + g1b_ref[...]
    nf2 = jnp.dot(a4, g2t_ref[...], preferred_element_type=f32) + g2b_ref[...]
    z1 = jnp.dot(a4, tf1_ref[...], preferred_element_type=f32) + c1_ref[...]
    z2 = jnp.dot(a4, tf2_ref[...], preferred_element_type=f32) + c2_ref[...]

    dn = (((1,), (1,)), ((), ()))
    ys = []
    for b in range(_G):
        r0 = b * _R
        nf1b = jax.lax.slice(nf1, (r0, 0), (r0 + _R, _D2))
        nf2b = jax.lax.slice(nf2, (r0, 0), (r0 + _R, _D2))
        z1b = jax.lax.slice(z1, (r0, 0), (r0 + _R, _HID))
        z2b = jax.lax.slice(z2, (r0, 0), (r0 + _R, _HID))
        g1 = jax.lax.dot_general(nf1b, nf1b, dn, preferred_element_type=f32)
        g2 = jax.lax.dot_general(nf2b, nf2b, dn, preferred_element_type=f32)

        s1 = _leaky(g1 + en1_ref[...]) + w1m_ref[...]
        e1 = jnp.exp(s1 - jnp.max(s1, axis=-1, keepdims=True))
        sm1 = e1 / jnp.sum(e1, axis=-1, keepdims=True)
        v1 = jnp.dot(sm1 * mx1_ref[...], z1b, preferred_element_type=f32) + z1b
        v1 = _leaky(v1 * m1s_ref[...] + m1b_ref[...])

        s2r = jnp.concatenate(
            [jax.lax.slice(g2, (64 * j, 0), (64 * j + 128, _R))
             for j in range(3)], axis=0)
        z2s = jnp.concatenate(
            [jax.lax.slice(z2b, (64 * j, 0), (64 * j + 128, _HID))
             for j in range(3)], axis=0)
        s2 = _leaky(s2r + dn2_ref[...]) + w2m_ref[...]
        e2 = jnp.exp(s2 - jnp.max(s2, axis=-1, keepdims=True))
        sm2 = e2 / jnp.sum(e2, axis=-1, keepdims=True)
        v2 = jnp.dot(sm2 * mx2_ref[...], z2b, preferred_element_type=f32) + z2s
        v2 = _leaky(v2 * m2s_ref[...] + m2b_ref[...])

        f = (jnp.sum(v1[:, :, None] * wp1_ref[...], axis=(0, 1))
             + jnp.sum(v2[:, :, None] * wp2_ref[...], axis=(0, 1)))[None, :]
        f = jnp.maximum(f + fb1_ref[...], 0.0)
        f = jnp.maximum(jnp.dot(f, f2t_ref[...], preferred_element_type=f32)
                        + fb2_ref[...], 0.0)
        f = jnp.maximum(jnp.dot(f, f3t_ref[...], preferred_element_type=f32)
                        + fb3_ref[...], 0.0)
        y = jnp.sum(f * f4_ref[...]) + fb4_ref[0, 0]
        ys.append(jnp.full((1, 8, 128), y, dtype=f32))
    out_ref[...] = jnp.concatenate(ys, axis=0)


@jax.jit
def kernel(X, params):
    p = params
    f32 = jnp.float32
    bs = X.shape[0]

    xu = jnp.transpose(X.reshape(bs, _TLEN, 16, _NN), (0, 1, 3, 2))
    xu = xu.reshape(bs * _TLEN * _NN, 16)

    ti = jnp.arange(16)[:, None] - jnp.arange(16)[None, :]
    bands = jnp.stack([(ti == k - 1).astype(f32) for k in range(3)])
    s_c1 = p['bn_c1_g'] / jnp.sqrt(1.0 + _EPS)
    w1c = p['conv1_w'][:, 0, :] * s_c1[:, None]
    m1 = jnp.einsum('ck,ktu->ctu', w1c, bands)
    m1 = jnp.transpose(m1, (1, 0, 2)).reshape(16, 128)
    b1 = jnp.repeat(p['bn_c1_b'], 16)[None, :]

    s_c2 = p['bn_c2_g'] / jnp.sqrt(1.0 + _EPS)
    w2c = p['conv2_w'] * s_c2[:, None, None]
    m2 = jnp.einsum('oik,ktu->itou', w2c, bands).reshape(128, 128)
    b2 = jnp.repeat(p['bn_c2_b'], 16)[None, :]

    s_l = p['bn2_g'] / jnp.sqrt(1.0 + _EPS)
    l2 = p['lin2_w'].T * s_l[None, :]
    l2b = (p['lin2_b'] * s_l + p['bn2_b'])[None, :]

    pos = jnp.arange(_TLEN, dtype=f32)[:, None]
    div = jnp.exp(jnp.arange(0, _D2, 2, dtype=f32) * (-np.log(10000.0) / _D2))
    pe = jnp.zeros((_TLEN, _D2), f32)
    pe = pe.at[:, 0::2].set(jnp.sin(pos * div))
    pe = pe.at[:, 1::2].set(jnp.cos(pos * div))
    pe_exp = jnp.tile(jnp.repeat(pe, _NN, axis=0), (_G, 1))

    def bn_s(g):
        return g / jnp.sqrt(1.0 + _EPS)

    t1t, t2t = p['t1_w'].T, p['t2_w'].T
    tf1 = bn_s(p['bnb1_g'])[:, None] * t1t
    c1 = (p['bnb1_b'] @ t1t + p['t1_b'])[None, :]
    tf2 = bn_s(p['bnb2_g'])[:, None] * t2t
    c2 = (p['bnb2_b'] @ t2t + p['t2_b'])[None, :]
    m1s, m1b = bn_s(p['bnm1_g'])[None, :], p['bnm1_b'][None, :]
    m2s, m2b = bn_s(p['bnm2_g'])[None, :], p['bnm2_b'][None, :]

    r = jnp.arange(_R)
    win1 = r // 64
    inw1 = (win1[:, None] == win1[None, :]).astype(f32)
    eye = jnp.eye(_R, dtype=f32)
    en1 = -1e8 * eye
    w1m = (inw1 - 1.0) * 1e9
    pat = r // 16
    dec = _DECAY ** jnp.abs(pat[:, None] - pat[None, :]).astype(f32)
    mx1 = dec * inw1

    RR = jnp.arange(384)
    j2 = RR // 128
    rl = RR % 128
    col = jnp.arange(_R)[None, :]
    diagcol = 64 * j2 + rl
    dn2 = jnp.where(col == diagcol[:, None], -1e8, 0.0).astype(f32)
    inw2 = ((col >= (64 * j2)[:, None]) & (col < (64 * j2 + 128)[:, None]))
    w2m = jnp.where(inw2, 0.0, -1e9).astype(f32)
    pat_r = rl // 16
    pat_c = (col - (64 * j2)[:, None]) // 16
    dec2 = _DECAY ** jnp.abs(pat_r[:, None] - pat_c).astype(f32)
    mx2 = jnp.where(inw2, dec2, 0.0).astype(f32)

    w1full = jnp.transpose(p['fc1_w'].reshape(_D2, 7 * _NN, _HID), (1, 2, 0))
    w1a, w1b_ = w1full[:64], w1full[64:]
    idx1 = (jnp.arange(_R) // 64) * 16 + (jnp.arange(_R) % 16)
    wp1 = w1a[idx1] * 0.25
    idx2 = (jnp.arange(384) // 128) * 16 + (jnp.arange(384) % 16)
    wp2 = w1b_[idx2] * 0.125
    fb1 = p['fc1_b'][None, :]
    f2t, fb2 = p['fc2_w'].T, p['fc2_b'][None, :]
    f3t, fb3 = p['fc3_w'].T, p['fc3_b'][None, :]
    f4 = p['fc4_w'][0][None, :]
    fb4 = p['fc4_b'][None, :]

    full = lambda shp: pl.BlockSpec(shp, lambda b: tuple(0 for _ in shp))
    in_specs = [
        pl.BlockSpec((_G * _R, 16), lambda b: (b, 0)),
        full((_G * _R, 16)),
        full((16, 128)), full((1, 128)),
        full((128, 128)), full((1, 128)),
        full((128, 16)), full((1, 16)),
        full((16, 16)), full((1, 16)),
        full((16, 16)), full((1, 16)),
        full((16, 8)), full((1, 8)),
        full((16, 8)), full((1, 8)),
        full((1, 8)), full((1, 8)),
        full((1, 8)), full((1, 8)),
        full((_R, _R)), full((_R, _R)), full((_R, _R)),
        full((384, _R)), full((384, _R)), full((384, _R)),
        full((_R, 8, 16)), full((384, 8, 16)), full((1, 16)),
        full((16, 16)), full((1, 16)),
        full((16, 8)), full((1, 8)),
        full((1, 8)), full((1, 1)),
    ]
    out = pl.pallas_call(
        _body,
        grid=(bs // _G,),
        in_specs=in_specs,
        out_specs=pl.BlockSpec((_G, 8, 128), lambda b: (b, 0, 0)),
        out_shape=jax.ShapeDtypeStruct((bs, 8, 128), f32),
        compiler_params=pltpu.CompilerParams(
            dimension_semantics=("arbitrary",),
        ),
    )(xu, pe_exp, m1, b1, m2, b2, l2, l2b,
      p['g1_w'].T, p['g1_b'][None, :],
      p['g2_w'].T, p['g2_b'][None, :],
      tf1, c1, tf2, c2, m1s, m1b, m2s, m2b,
      en1, w1m, mx1, dn2, w2m, mx2,
      wp1, wp2, fb1, f2t, fb2, f3t, fb3, f4, fb4)
    return out[:, 0, :1]


# speedup vs baseline: 3.0454x; 3.0454x over previous
import jax
import jax.numpy as jnp
import numpy as np
from jax.experimental import pallas as pl
from jax.experimental.pallas import tpu as pltpu

_G = 4
_TLEN = 16
_NN = 16
_D2 = 16
_HID = 8
_EPS = 1e-5
_DECAY = 0.7
_NEG = 0.01
_R = 256


def _leaky(x):
    return jnp.where(x >= 0, x, _NEG * x)


def _body(xu_ref, pe_ref, m1_ref, b1_ref, m2_ref, b2_ref, l2_ref, l2b_ref,
          g1t_ref, g1b_ref, g2t_ref, g2b_ref,
          tf1_ref, c1_ref, tf2_ref, c2_ref,
          m1s_ref, m1b_ref, m2s_ref, m2b_ref,
          mx1_ref, iw1_ref, mx2_ref, iw2_ref,
          wp_ref, fb1_ref,
          f2t_ref, fb2_ref, f3t_ref, fb3_ref, f4_ref, fb4_ref,
          out_ref):
    f32 = jnp.float32
    x = xu_ref[...]
    h = jnp.maximum(jnp.dot(x, m1_ref[...], preferred_element_type=f32)
                    + b1_ref[...], 0.0)
    h = jnp.maximum(jnp.dot(h, m2_ref[...], preferred_element_type=f32)
                    + b2_ref[...], 0.0)
    a4 = (jnp.dot(h, l2_ref[...], preferred_element_type=f32)
          + l2b_ref[...] + pe_ref[...])

    nf1 = jnp.dot(a4, g1t_ref[...], preferred_element_type=f32) + g1b_ref[...]
    nf2 = jnp.dot(a4, g2t_ref[...], preferred_element_type=f32) + g2b_ref[...]
    z1 = jnp.dot(a4, tf1_ref[...], preferred_element_type=f32) + c1_ref[...]
    z2 = jnp.dot(a4, tf2_ref[...], preferred_element_type=f32) + c2_ref[...]

    dn = (((1,), (1,)), ((), ()))
    dn0 = (((0,), (0,)), ((), ()))
    fs = []
    for b in range(_G):
        r0 = b * _R
        nf1b = jax.lax.slice(nf1, (r0, 0), (r0 + _R, _D2))
        nf2b = jax.lax.slice(nf2, (r0, 0), (r0 + _R, _D2))
        z1b = jax.lax.slice(z1, (r0, 0), (r0 + _R, _HID))
        z2b = jax.lax.slice(z2, (r0, 0), (r0 + _R, _HID))
        g1 = jax.lax.dot_general(nf1b, nf1b, dn, preferred_element_type=f32)
        g2 = jax.lax.dot_general(nf2b, nf2b, dn, preferred_element_type=f32)

        e1 = jnp.exp(_leaky(g1))
        s1 = jnp.sum(e1 * iw1_ref[...], axis=-1, keepdims=True)
        v1 = (jnp.dot(e1 * mx1_ref[...], z1b, preferred_element_type=f32) / s1
              + z1b)
        v1 = _leaky(v1 * m1s_ref[...] + m1b_ref[...])

        s2r = jnp.concatenate(
            [jax.lax.slice(g2, (64 * j, 0), (64 * j + 128, _R))
             for j in range(3)], axis=0)
        z2s = jnp.concatenate(
            [jax.lax.slice(z2b, (64 * j, 0), (64 * j + 128, _HID))
             for j in range(3)], axis=0)
        e2 = jnp.exp(_leaky(s2r))
        s2 = jnp.sum(e2 * iw2_ref[...], axis=-1, keepdims=True)
        v2 = (jnp.dot(e2 * mx2_ref[...], z2b, preferred_element_type=f32) / s2
              + z2s)
        v2 = _leaky(v2 * m2s_ref[...] + m2b_ref[...])

        v = jnp.concatenate([v1, v2], axis=0)
        c = jax.lax.dot_general(v, wp_ref[...], dn0,
                                preferred_element_type=f32)
        f = jax.lax.slice(c, (0, 0), (1, 16))
        for hh in range(1, _HID):
            f = f + jax.lax.slice(c, (hh, 16 * hh), (hh + 1, 16 * hh + 16))
        fs.append(f)

    f = jnp.concatenate(fs, axis=0)
    f = jnp.maximum(f + fb1_ref[...], 0.0)
    f = jnp.maximum(jnp.dot(f, f2t_ref[...], preferred_element_type=f32)
                    + fb2_ref[...], 0.0)
    f = jnp.maximum(jnp.dot(f, f3t_ref[...], preferred_element_type=f32)
                    + fb3_ref[...], 0.0)
    y = jnp.sum(f * f4_ref[...], axis=-1) + fb4_ref[0, 0]
    out_ref[...] = jnp.broadcast_to(y[:, None, None], (_G, 8, 128))


@jax.jit
def kernel(X, params):
    p = params
    f32 = jnp.float32
    bs = X.shape[0]

    xu = jnp.transpose(X.reshape(bs, _TLEN, 16, _NN), (0, 1, 3, 2))
    xu = xu.reshape(bs * _TLEN * _NN, 16)

    ti = jnp.arange(16)[:, None] - jnp.arange(16)[None, :]
    bands = jnp.stack([(ti == k - 1).astype(f32) for k in range(3)])
    s_c1 = p['bn_c1_g'] / jnp.sqrt(1.0 + _EPS)
    w1c = p['conv1_w'][:, 0, :] * s_c1[:, None]
    m1 = jnp.einsum('ck,ktu->ctu', w1c, bands)
    m1 = jnp.transpose(m1, (1, 0, 2)).reshape(16, 128)
    b1 = jnp.repeat(p['bn_c1_b'], 16)[None, :]

    s_c2 = p['bn_c2_g'] / jnp.sqrt(1.0 + _EPS)
    w2c = p['conv2_w'] * s_c2[:, None, None]
    m2 = jnp.einsum('oik,ktu->itou', w2c, bands).reshape(128, 128)
    b2 = jnp.repeat(p['bn_c2_b'], 16)[None, :]

    s_l = p['bn2_g'] / jnp.sqrt(1.0 + _EPS)
    l2 = p['lin2_w'].T * s_l[None, :]
    l2b = (p['lin2_b'] * s_l + p['bn2_b'])[None, :]

    pos = jnp.arange(_TLEN, dtype=f32)[:, None]
    div = jnp.exp(jnp.arange(0, _D2, 2, dtype=f32) * (-np.log(10000.0) / _D2))
    pe = jnp.zeros((_TLEN, _D2), f32)
    pe = pe.at[:, 0::2].set(jnp.sin(pos * div))
    pe = pe.at[:, 1::2].set(jnp.cos(pos * div))
    pe_exp = jnp.tile(jnp.repeat(pe, _NN, axis=0), (_G, 1))

    def bn_s(g):
        return g / jnp.sqrt(1.0 + _EPS)

    t1t, t2t = p['t1_w'].T, p['t2_w'].T
    tf1 = bn_s(p['bnb1_g'])[:, None] * t1t
    c1 = (p['bnb1_b'] @ t1t + p['t1_b'])[None, :]
    tf2 = bn_s(p['bnb2_g'])[:, None] * t2t
    c2 = (p['bnb2_b'] @ t2t + p['t2_b'])[None, :]
    m1s, m1b = bn_s(p['bnm1_g'])[None, :], p['bnm1_b'][None, :]
    m2s, m2b = bn_s(p['bnm2_g'])[None, :], p['bnm2_b'][None, :]

    r = jnp.arange(_R)
    win1 = r // 64
    inw1 = (win1[:, None] == win1[None, :]).astype(f32)
    eye = jnp.eye(_R, dtype=f32)
    pat = r // 16
    dec = _DECAY ** jnp.abs(pat[:, None] - pat[None, :]).astype(f32)
    mx1 = dec * inw1 - eye
    iw1 = inw1 - eye

    RR = jnp.arange(384)
    j2 = RR // 128
    rl = RR % 128
    col = jnp.arange(_R)[None, :]
    diagcol = (64 * j2 + rl)[:, None]
    inw2 = ((col >= (64 * j2)[:, None]) & (col < (64 * j2 + 128)[:, None]))
    pat_c = (col - (64 * j2)[:, None]) // 16
    dec2 = _DECAY ** jnp.abs((rl // 16)[:, None] - pat_c).astype(f32)
    ond = (col == diagcol)
    mx2 = jnp.where(inw2 & ~ond, dec2, 0.0).astype(f32)
    iw2 = jnp.where(inw2 & ~ond, 1.0, 0.0).astype(f32)

    w1full = jnp.transpose(p['fc1_w'].reshape(_D2, 7 * _NN, _HID), (1, 2, 0))
    w1a, w1b_ = w1full[:64], w1full[64:]
    idx1 = (jnp.arange(_R) // 64) * 16 + (jnp.arange(_R) % 16)
    wp1 = (w1a[idx1] * 0.25).reshape(_R, 128)
    idx2 = (jnp.arange(384) // 128) * 16 + (jnp.arange(384) % 16)
    wp2 = (w1b_[idx2] * 0.125).reshape(384, 128)
    wp = jnp.concatenate([wp1, wp2], axis=0)
    fb1 = p['fc1_b'][None, :]
    f2t, fb2 = p['fc2_w'].T, p['fc2_b'][None, :]
    f3t, fb3 = p['fc3_w'].T, p['fc3_b'][None, :]
    f4 = p['fc4_w'][0][None, :]
    fb4 = p['fc4_b'][None, :]

    full = lambda shp: pl.BlockSpec(shp, lambda b: tuple(0 for _ in shp))
    in_specs = [
        pl.BlockSpec((_G * _R, 16), lambda b: (b, 0)),
        full((_G * _R, 16)),
        full((16, 128)), full((1, 128)),
        full((128, 128)), full((1, 128)),
        full((128, 16)), full((1, 16)),
        full((16, 16)), full((1, 16)),
        full((16, 16)), full((1, 16)),
        full((16, 8)), full((1, 8)),
        full((16, 8)), full((1, 8)),
        full((1, 8)), full((1, 8)),
        full((1, 8)), full((1, 8)),
        full((_R, _R)), full((_R, _R)),
        full((384, _R)), full((384, _R)),
        full((640, 128)), full((1, 16)),
        full((16, 16)), full((1, 16)),
        full((16, 8)), full((1, 8)),
        full((1, 8)), full((1, 1)),
    ]
    out = pl.pallas_call(
        _body,
        grid=(bs // _G,),
        in_specs=in_specs,
        out_specs=pl.BlockSpec((_G, 8, 128), lambda b: (b, 0, 0)),
        out_shape=jax.ShapeDtypeStruct((bs, 8, 128), f32),
        compiler_params=pltpu.CompilerParams(
            dimension_semantics=("arbitrary",),
        ),
    )(xu, pe_exp, m1, b1, m2, b2, l2, l2b,
      p['g1_w'].T, p['g1_b'][None, :],
      p['g2_w'].T, p['g2_b'][None, :],
      tf1, c1, tf2, c2, m1s, m1b, m2s, m2b,
      mx1, iw1, mx2, iw2,
      wp, fb1, f2t, fb2, f3t, fb3, f4, fb4)
    return out[:, 0, :1]
